# baseline (speedup 1.0000x reference)
import functools

import jax
import jax.numpy as jnp
from jax.experimental import pallas as pl
from jax.experimental.pallas import tpu as pltpu

B = 32
F = 64
BM = 512


def _score_copy_body(x_ref, w1a_ref, w1b_ref, b1_ref, w2_ref, thr_ref,
                     mem_ref, out_mem_ref, vals_ref, slots_ref, flag_ref,
                     m_s, s_s, idx_s):
    i = pl.program_id(0)
    nblk = pl.num_programs(0)

    @pl.when(i == 0)
    def _init():
        m_s[...] = jnp.full((B, 1), -jnp.inf, jnp.float32)
        s_s[...] = jnp.zeros((B, 1), jnp.float32)
        idx_s[...] = jnp.zeros((B, 1), jnp.int32)

    x = x_ref[...]
    in_proj = jnp.dot(x, w1a_ref[...],
                      preferred_element_type=jnp.float32)
    memb = mem_ref[...]
    out_mem_ref[...] = memb.reshape(BM, 1, F)
    mem_proj = jnp.dot(memb, w1b_ref[...],
                       preferred_element_type=jnp.float32)

    h = jnp.tanh(in_proj[:, None, :] + mem_proj[None, :, :]
                 + b1_ref[...][None, :, :])
    scores = jnp.dot(h.reshape(B * BM, F), w2_ref[...],
                     preferred_element_type=jnp.float32).reshape(B, BM)

    blk_max = jnp.max(scores, axis=1, keepdims=True)
    blk_arg = (jnp.argmax(scores, axis=1).astype(jnp.int32).reshape(B, 1)
               + i * BM)
    m_old = m_s[...]
    m_new = jnp.maximum(m_old, blk_max)
    s_new = (s_s[...] * jnp.exp(m_old - m_new)
             + jnp.sum(jnp.exp(scores - m_new), axis=1, keepdims=True))
    idx_s[...] = jnp.where(blk_max > m_old, blk_arg, idx_s[...])
    m_s[...] = m_new
    s_s[...] = s_new

    @pl.when(i == nblk - 1)
    def _finalize():
        best_w = 1.0 / s_s[...]
        do_write = best_w > thr_ref[...]
        slot = idx_s[...]
        eq = slot == slot.reshape(1, B)
        ii = jax.lax.broadcasted_iota(jnp.int32, (B, B), 0)
        jj = jax.lax.broadcasted_iota(jnp.int32, (B, B), 1)
        conflict = jnp.any(eq & (jj > ii) & do_write.reshape(1, B),
                           axis=1, keepdims=True)
        final_write = do_write & jnp.logical_not(conflict)
        flag = jnp.any(final_write)
        ivec = jax.lax.broadcasted_iota(jnp.int32, (B, 1), 0)
        k = jnp.maximum(jnp.max(jnp.where(final_write, ivec, -1)), 0)
        onehot_k = ivec == k
        slot_k = jnp.sum(jnp.where(onehot_k, slot, 0))
        val_k = jnp.sum(jnp.where(onehot_k, x, 0.0), axis=0, keepdims=True)
        slots_ref[...] = jnp.where(final_write, slot, slot_k).reshape(1, B)
        vals_ref[...] = jnp.where(final_write, x, val_k)
        flag_ref[...] = flag.astype(jnp.int32).reshape(1, 1)


def _scatter_body(slots_ref, flag_ref, vals_ref, row_ref, out_ref):
    out_ref[...] = jnp.where(flag_ref[0] != 0, vals_ref[...], row_ref[...])


@functools.partial(jax.jit, static_argnames=())
def kernel(input_data, memory, W1, b1, W2, b2, threshold):
    del b2
    M = memory.shape[0]
    nblk = M // BM

    w1a = W1[:F, :]
    w1b = W1[F:, :]
    b1r = b1.reshape(1, F)
    thr = threshold.reshape(1, 1)

    out_mem, vals, slots, flag = pl.pallas_call(
        _score_copy_body,
        grid=(nblk,),
        in_specs=[
            pl.BlockSpec((B, F), lambda i: (0, 0)),
            pl.BlockSpec((F, F), lambda i: (0, 0)),
            pl.BlockSpec((F, F), lambda i: (0, 0)),
            pl.BlockSpec((1, F), lambda i: (0, 0)),
            pl.BlockSpec((F, 1), lambda i: (0, 0)),
            pl.BlockSpec((1, 1), lambda i: (0, 0)),
            pl.BlockSpec((BM, F), lambda i: (i, 0)),
        ],
        out_specs=[
            pl.BlockSpec((BM, 1, F), lambda i: (i, 0, 0)),
            pl.BlockSpec((B, F), lambda i: (0, 0)),
            pl.BlockSpec((1, B), lambda i: (0, 0)),
            pl.BlockSpec((1, 1), lambda i: (0, 0)),
        ],
        out_shape=[
            jax.ShapeDtypeStruct((M, 1, F), jnp.float32),
            jax.ShapeDtypeStruct((B, F), jnp.float32),
            jax.ShapeDtypeStruct((1, B), jnp.int32),
            jax.ShapeDtypeStruct((1, 1), jnp.int32),
        ],
        scratch_shapes=[
            pltpu.VMEM((B, 1), jnp.float32),
            pltpu.VMEM((B, 1), jnp.float32),
            pltpu.VMEM((B, 1), jnp.int32),
        ],
    )(input_data, w1a, w1b, b1r, W2, thr, memory)

    slots1d = slots.reshape(B)
    flag1d = flag.reshape(1)
    vals3 = vals.reshape(B, 1, F)

    grid_spec = pltpu.PrefetchScalarGridSpec(
        num_scalar_prefetch=2,
        grid=(B,),
        in_specs=[
            pl.BlockSpec((1, 1, F), lambda i, slots, flag: (i, 0, 0)),
            pl.BlockSpec((1, 1, F), lambda i, slots, flag: (slots[i], 0, 0)),
        ],
        out_specs=pl.BlockSpec((1, 1, F), lambda i, slots, flag: (slots[i], 0, 0)),
    )
    updated = pl.pallas_call(
        _scatter_body,
        grid_spec=grid_spec,
        out_shape=jax.ShapeDtypeStruct((M, 1, F), jnp.float32),
        input_output_aliases={3: 0},
    )(slots1d, flag1d, vals3, out_mem)
    return updated.reshape(M, F)


# speedup vs baseline: 2.9126x; 2.9126x over previous
import functools

import jax
import jax.numpy as jnp
from jax.experimental import pallas as pl
from jax.experimental.pallas import tpu as pltpu

B = 32
F = 64
BM = 512


def _score_copy_body(x_ref, w1a_ref, w1bt_ref, b1_ref, w2_ref, thr_ref,
                     mem_ref, out_mem_ref, vals_ref, slots_ref, flag_ref,
                     m_s, s_s, idx_s):
    i = pl.program_id(0)
    nblk = pl.num_programs(0)

    @pl.when(i == 0)
    def _init():
        m_s[...] = jnp.full((B, 1), -jnp.inf, jnp.float32)
        s_s[...] = jnp.zeros((B, 1), jnp.float32)
        idx_s[...] = jnp.zeros((B, 1), jnp.int32)

    x = x_ref[...]
    in_proj = jnp.dot(x, w1a_ref[...],
                      preferred_element_type=jnp.float32) + b1_ref[...]
    memb = mem_ref[...]
    out_mem_ref[...] = memb
    mem_projT = jax.lax.dot_general(
        w1bt_ref[...], memb, (((1,), (1,)), ((), ())),
        preferred_element_type=jnp.float32)

    h = jnp.tanh(mem_projT[None, :, :] + in_proj[:, :, None])
    scores = jnp.sum(h * w2_ref[...][None, :, :], axis=1)

    blk_max = jnp.max(scores, axis=1, keepdims=True)
    blk_arg = (jnp.argmax(scores, axis=1).astype(jnp.int32).reshape(B, 1)
               + i * BM)
    m_old = m_s[...]
    m_new = jnp.maximum(m_old, blk_max)
    s_new = (s_s[...] * jnp.exp(m_old - m_new)
             + jnp.sum(jnp.exp(scores - m_new), axis=1, keepdims=True))
    idx_s[...] = jnp.where(blk_max > m_old, blk_arg, idx_s[...])
    m_s[...] = m_new
    s_s[...] = s_new

    @pl.when(i == nblk - 1)
    def _finalize():
        best_w = 1.0 / s_s[...]
        do_write = best_w > thr_ref[...]
        slot = idx_s[...]
        eq = slot == slot.reshape(1, B)
        ii = jax.lax.broadcasted_iota(jnp.int32, (B, B), 0)
        jj = jax.lax.broadcasted_iota(jnp.int32, (B, B), 1)
        conflict = jnp.any(eq & (jj > ii) & do_write.reshape(1, B),
                           axis=1, keepdims=True)
        final_write = do_write & jnp.logical_not(conflict)
        flag = jnp.any(final_write)
        ivec = jax.lax.broadcasted_iota(jnp.int32, (B, 1), 0)
        k = jnp.maximum(jnp.max(jnp.where(final_write, ivec, -1)), 0)
        onehot_k = ivec == k
        slot_k = jnp.sum(jnp.where(onehot_k, slot, 0))
        val_k = jnp.sum(jnp.where(onehot_k, x, 0.0), axis=0, keepdims=True)
        slots_ref[...] = jnp.where(final_write, slot, slot_k).reshape(1, B)
        vals_ref[...] = jnp.where(final_write, x, val_k)
        flag_ref[...] = flag.astype(jnp.int32).reshape(1, 1)


def _scatter_body(slots_ref, flag_ref, vals_ref, row_ref, out_ref):
    out_ref[...] = jnp.where(flag_ref[0] != 0, vals_ref[...], row_ref[...])


@functools.partial(jax.jit, static_argnames=())
def kernel(input_data, memory, W1, b1, W2, b2, threshold):
    del b2
    M = memory.shape[0]
    nblk = M // BM

    w1a = W1[:F, :]
    w1bt = W1[F:, :].T
    b1r = b1.reshape(1, F)
    thr = threshold.reshape(1, 1)

    out_mem, vals, slots, flag = pl.pallas_call(
        _score_copy_body,
        grid=(nblk,),
        in_specs=[
            pl.BlockSpec((B, F), lambda i: (0, 0)),
            pl.BlockSpec((F, F), lambda i: (0, 0)),
            pl.BlockSpec((F, F), lambda i: (0, 0)),
            pl.BlockSpec((1, F), lambda i: (0, 0)),
            pl.BlockSpec((F, 1), lambda i: (0, 0)),
            pl.BlockSpec((1, 1), lambda i: (0, 0)),
            pl.BlockSpec((BM, F), lambda i: (i, 0)),
        ],
        out_specs=[
            pl.BlockSpec((BM, F), lambda i: (i, 0)),
            pl.BlockSpec((B, F), lambda i: (0, 0)),
            pl.BlockSpec((1, B), lambda i: (0, 0)),
            pl.BlockSpec((1, 1), lambda i: (0, 0)),
        ],
        out_shape=[
            jax.ShapeDtypeStruct((M, F), jnp.float32),
            jax.ShapeDtypeStruct((B, F), jnp.float32),
            jax.ShapeDtypeStruct((1, B), jnp.int32),
            jax.ShapeDtypeStruct((1, 1), jnp.int32),
        ],
        scratch_shapes=[
            pltpu.VMEM((B, 1), jnp.float32),
            pltpu.VMEM((B, 1), jnp.float32),
            pltpu.VMEM((B, 1), jnp.int32),
        ],
    )(input_data, w1a, w1bt, b1r, W2, thr, memory)

    slots1d = slots.reshape(B)
    flag1d = flag.reshape(1)
    vals3 = vals.reshape(B, 1, F)
    out3 = out_mem.reshape(M, 1, F)

    grid_spec = pltpu.PrefetchScalarGridSpec(
        num_scalar_prefetch=2,
        grid=(B,),
        in_specs=[
            pl.BlockSpec((1, 1, F), lambda i, slots, flag: (i, 0, 0)),
            pl.BlockSpec((1, 1, F), lambda i, slots, flag: (slots[i], 0, 0)),
        ],
        out_specs=pl.BlockSpec((1, 1, F), lambda i, slots, flag: (slots[i], 0, 0)),
    )
    updated = pl.pallas_call(
        _scatter_body,
        grid_spec=grid_spec,
        out_shape=jax.ShapeDtypeStruct((M, 1, F), jnp.float32),
        input_output_aliases={3: 0},
    )(slots1d, flag1d, vals3, out3)
    return updated.reshape(M, F)


# speedup vs baseline: 3.2954x; 1.1314x over previous
import functools

import jax
import jax.numpy as jnp
from jax.experimental import pallas as pl
from jax.experimental.pallas import tpu as pltpu

B = 32
F = 64
BM = 512


def _score_copy_body(x_ref, w1a_ref, w1bt_ref, b1_ref, w2_ref, thr_ref,
                     mem_ref, out_mem_ref, vals_ref, slots_ref, flag_ref,
                     m_s, s_s, idx_s):
    i = pl.program_id(0)
    nblk = pl.num_programs(0)

    @pl.when(i == 0)
    def _init():
        m_s[...] = jnp.full((B, 1), -jnp.inf, jnp.float32)
        s_s[...] = jnp.zeros((B, 1), jnp.float32)
        idx_s[...] = jnp.zeros((B, 1), jnp.int32)

    x = x_ref[...]
    in_proj = jnp.dot(x, w1a_ref[...],
                      preferred_element_type=jnp.float32) + b1_ref[...]
    memb = mem_ref[...]
    out_mem_ref[...] = memb
    mem_projT = jax.lax.dot_general(
        w1bt_ref[...], memb, (((1,), (1,)), ((), ())),
        preferred_element_type=jnp.float32)

    in_projT = in_proj.T
    w2row = w2_ref[...]
    rows = []
    for b in range(B):
        argb = mem_projT + in_projT[:, b:b + 1]
        rows.append(jnp.dot(w2row, jnp.tanh(argb),
                            preferred_element_type=jnp.float32))
    scores = jnp.concatenate(rows, axis=0)

    blk_max = jnp.max(scores, axis=1, keepdims=True)
    blk_arg = (jnp.argmax(scores, axis=1).astype(jnp.int32).reshape(B, 1)
               + i * BM)
    m_old = m_s[...]
    m_new = jnp.maximum(m_old, blk_max)
    s_new = (s_s[...] * jnp.exp(m_old - m_new)
             + jnp.sum(jnp.exp(scores - m_new), axis=1, keepdims=True))
    idx_s[...] = jnp.where(blk_max > m_old, blk_arg, idx_s[...])
    m_s[...] = m_new
    s_s[...] = s_new

    @pl.when(i == nblk - 1)
    def _finalize():
        best_w = 1.0 / s_s[...]
        do_write = best_w > thr_ref[...]
        slot = idx_s[...]
        eq = slot == slot.reshape(1, B)
        ii = jax.lax.broadcasted_iota(jnp.int32, (B, B), 0)
        jj = jax.lax.broadcasted_iota(jnp.int32, (B, B), 1)
        conflict = jnp.any(eq & (jj > ii) & do_write.reshape(1, B),
                           axis=1, keepdims=True)
        final_write = do_write & jnp.logical_not(conflict)
        flag = jnp.any(final_write)
        ivec = jax.lax.broadcasted_iota(jnp.int32, (B, 1), 0)
        k = jnp.maximum(jnp.max(jnp.where(final_write, ivec, -1)), 0)
        onehot_k = ivec == k
        slot_k = jnp.sum(jnp.where(onehot_k, slot, 0))
        val_k = jnp.sum(jnp.where(onehot_k, x, 0.0), axis=0, keepdims=True)
        slots_ref[...] = jnp.where(final_write, slot, slot_k).reshape(1, B)
        vals_ref[...] = jnp.where(final_write, x, val_k)
        flag_ref[...] = flag.astype(jnp.int32).reshape(1, 1)


def _scatter_body(slots_ref, flag_ref, vals_ref, row_ref, out_ref):
    out_ref[...] = jnp.where(flag_ref[0] != 0, vals_ref[...], row_ref[...])


@functools.partial(jax.jit, static_argnames=())
def kernel(input_data, memory, W1, b1, W2, b2, threshold):
    del b2
    M = memory.shape[0]
    nblk = M // BM

    w1a = W1[:F, :]
    w1bt = W1[F:, :].T
    b1r = b1.reshape(1, F)
    thr = threshold.reshape(1, 1)

    out_mem, vals, slots, flag = pl.pallas_call(
        _score_copy_body,
        grid=(nblk,),
        in_specs=[
            pl.BlockSpec((B, F), lambda i: (0, 0)),
            pl.BlockSpec((F, F), lambda i: (0, 0)),
            pl.BlockSpec((F, F), lambda i: (0, 0)),
            pl.BlockSpec((1, F), lambda i: (0, 0)),
            pl.BlockSpec((1, F), lambda i: (0, 0)),
            pl.BlockSpec((1, 1), lambda i: (0, 0)),
            pl.BlockSpec((BM, F), lambda i: (i, 0)),
        ],
        out_specs=[
            pl.BlockSpec((BM, F), lambda i: (i, 0)),
            pl.BlockSpec((B, F), lambda i: (0, 0)),
            pl.BlockSpec((1, B), lambda i: (0, 0)),
            pl.BlockSpec((1, 1), lambda i: (0, 0)),
        ],
        out_shape=[
            jax.ShapeDtypeStruct((M, F), jnp.float32),
            jax.ShapeDtypeStruct((B, F), jnp.float32),
            jax.ShapeDtypeStruct((1, B), jnp.int32),
            jax.ShapeDtypeStruct((1, 1), jnp.int32),
        ],
        scratch_shapes=[
            pltpu.VMEM((B, 1), jnp.float32),
            pltpu.VMEM((B, 1), jnp.float32),
            pltpu.VMEM((B, 1), jnp.int32),
        ],
    )(input_data, w1a, w1bt, b1r, W2.reshape(1, F), thr, memory)

    slots1d = slots.reshape(B)
    flag1d = flag.reshape(1)
    vals3 = vals.reshape(B, 1, F)
    out3 = out_mem.reshape(M, 1, F)

    grid_spec = pltpu.PrefetchScalarGridSpec(
        num_scalar_prefetch=2,
        grid=(B,),
        in_specs=[
            pl.BlockSpec((1, 1, F), lambda i, slots, flag: (i, 0, 0)),
            pl.BlockSpec((1, 1, F), lambda i, slots, flag: (slots[i], 0, 0)),
        ],
        out_specs=pl.BlockSpec((1, 1, F), lambda i, slots, flag: (slots[i], 0, 0)),
    )
    updated = pl.pallas_call(
        _scatter_body,
        grid_spec=grid_spec,
        out_shape=jax.ShapeDtypeStruct((M, 1, F), jnp.float32),
        input_output_aliases={3: 0},
    )(slots1d, flag1d, vals3, out3)
    return updated.reshape(M, F)
